# baseline (speedup 1.0000x reference)
import jax
import jax.numpy as jnp
from jax.experimental import pallas as pl
from jax.experimental.pallas import tpu as pltpu

_B = 32
_C = 64
_P = 128
_L = 16
_COLS = _P
_ROWS = _B * _C * _L
_CH = 8192
_NCH = _ROWS // _CH
_PAT = 8192
_NPAT = _ROWS // _PAT


def _mask_kernel(x_hbm, moe_hbm, meo_hbm, oe_hbm, eo_hbm,
                 xb0, xb1, xb2, xb3, mo0, mo1, me0, me1, oe_pat, eo_pat,
                 sem_in, sem_moe, sem_meo, sem_mask):
    xbufs = (xb0, xb1, xb2, xb3)
    mobufs = (mo0, mo1)
    mebufs = (me0, me1)

    def in_copy(c):
        return pltpu.make_async_copy(
            x_hbm.at[pl.ds(c * _CH, _CH), :], xbufs[c], sem_in)

    def moe_copy(c):
        return pltpu.make_async_copy(
            mobufs[c % 2], moe_hbm.at[pl.ds(c * _CH, _CH), :], sem_moe)

    def meo_copy(c):
        return pltpu.make_async_copy(
            mebufs[c % 2], meo_hbm.at[pl.ds(c * _CH, _CH), :], sem_meo)

    for c in range(_NCH):
        in_copy(c).start()

    prow = jax.lax.broadcasted_iota(jnp.int32, (_PAT, _COLS), 0)
    pcol = jax.lax.broadcasted_iota(jnp.int32, (_PAT, _COLS), 1)
    poe = ((prow // _L) ^ pcol) & 1
    oe_pat[...] = poe
    for k in range(_NPAT):
        pltpu.make_async_copy(
            oe_pat, oe_hbm.at[pl.ds(k * _PAT, _PAT), :], sem_mask).start()
    eo_pat[...] = poe ^ 1
    for k in range(_NPAT):
        pltpu.make_async_copy(
            eo_pat, eo_hbm.at[pl.ds(k * _PAT, _PAT), :], sem_mask).start()

    for c in range(_NCH):
        in_copy(c).wait()
        if c >= 2:
            moe_copy(c - 2).wait()
            meo_copy(c - 2).wait()
        x = xbufs[c][...]
        crow = jax.lax.broadcasted_iota(jnp.int32, (_CH, _COLS), 0)
        ccol = jax.lax.broadcasted_iota(jnp.int32, (_CH, _COLS), 1)
        keep_oe = (((crow // _L) ^ ccol) & 1) == 0
        zero = jnp.zeros_like(x)
        mobufs[c % 2][...] = jnp.where(keep_oe, x, zero)
        mebufs[c % 2][...] = jnp.where(keep_oe, zero, x)
        moe_copy(c).start()
        meo_copy(c).start()

    moe_copy(_NCH - 2).wait()
    meo_copy(_NCH - 2).wait()
    moe_copy(_NCH - 1).wait()
    meo_copy(_NCH - 1).wait()
    for k in range(_NPAT):
        pltpu.make_async_copy(
            oe_pat, oe_hbm.at[pl.ds(k * _PAT, _PAT), :], sem_mask).wait()
        pltpu.make_async_copy(
            eo_pat, eo_hbm.at[pl.ds(k * _PAT, _PAT), :], sem_mask).wait()


def kernel(inputs):
    x2d = jnp.transpose(inputs, (0, 1, 3, 2)).reshape(_ROWS, _COLS)
    any_spec = pl.BlockSpec(memory_space=pl.ANY)
    out = pl.pallas_call(
        _mask_kernel,
        in_specs=[any_spec],
        out_specs=[any_spec, any_spec, any_spec, any_spec],
        out_shape=[
            jax.ShapeDtypeStruct((_ROWS, _COLS), jnp.float32),
            jax.ShapeDtypeStruct((_ROWS, _COLS), jnp.float32),
            jax.ShapeDtypeStruct((_ROWS, _COLS), jnp.int32),
            jax.ShapeDtypeStruct((_ROWS, _COLS), jnp.int32),
        ],
        scratch_shapes=[
            pltpu.VMEM((_CH, _COLS), jnp.float32),
            pltpu.VMEM((_CH, _COLS), jnp.float32),
            pltpu.VMEM((_CH, _COLS), jnp.float32),
            pltpu.VMEM((_CH, _COLS), jnp.float32),
            pltpu.VMEM((_CH, _COLS), jnp.float32),
            pltpu.VMEM((_CH, _COLS), jnp.float32),
            pltpu.VMEM((_CH, _COLS), jnp.float32),
            pltpu.VMEM((_CH, _COLS), jnp.float32),
            pltpu.VMEM((_PAT, _COLS), jnp.int32),
            pltpu.VMEM((_PAT, _COLS), jnp.int32),
            pltpu.SemaphoreType.DMA,
            pltpu.SemaphoreType.DMA,
            pltpu.SemaphoreType.DMA,
            pltpu.SemaphoreType.DMA,
        ],
    )(x2d)

    def _back(a):
        return jnp.transpose(a.reshape(_B, _C, _L, _P), (0, 1, 3, 2))

    return tuple(_back(a) for a in out)


# speedup vs baseline: 1.0287x; 1.0287x over previous
import jax
import jax.numpy as jnp
from jax.experimental import pallas as pl
from jax.experimental.pallas import tpu as pltpu

_B = 32
_C = 64
_P = 128
_L = 16
_COLS = _P
_ROWS = _B * _C * _L
_CH = 8192
_NCH = _ROWS // _CH
_PAT = 8192
_NPAT = _ROWS // _PAT


def _mask_kernel(x_hbm, moe_hbm, meo_hbm, oe_hbm, eo_hbm,
                 xb0, xb1, xb2, xb3, mo0, mo1, me0, me1, oe_pat, eo_pat,
                 sem_in, sem_moe, sem_meo, sem_mask):
    xbufs = (xb0, xb1, xb2, xb3)
    mobufs = (mo0, mo1)
    mebufs = (me0, me1)

    def in_copy(c):
        return pltpu.make_async_copy(
            x_hbm.at[pl.ds(c * _CH, _CH), :], xbufs[c], sem_in)

    def moe_copy(c):
        return pltpu.make_async_copy(
            mobufs[c % 2], moe_hbm.at[pl.ds(c * _CH, _CH), :], sem_moe)

    def meo_copy(c):
        return pltpu.make_async_copy(
            mebufs[c % 2], meo_hbm.at[pl.ds(c * _CH, _CH), :], sem_meo)

    for c in range(_NCH):
        in_copy(c).start()

    prow = jax.lax.broadcasted_iota(jnp.int32, (_PAT, _COLS), 0)
    pcol = jax.lax.broadcasted_iota(jnp.int32, (_PAT, _COLS), 1)
    poe = ((prow // _L) ^ pcol) & 1
    oe_pat[...] = poe
    eo_pat[...] = poe ^ 1
    for k in range(_NPAT):
        pltpu.make_async_copy(
            oe_pat, oe_hbm.at[pl.ds(k * _PAT, _PAT), :], sem_mask).start()
        pltpu.make_async_copy(
            eo_pat, eo_hbm.at[pl.ds(k * _PAT, _PAT), :], sem_mask).start()

    for c in range(_NCH):
        in_copy(c).wait()
        if c >= 2:
            moe_copy(c - 2).wait()
            meo_copy(c - 2).wait()
        x = xbufs[c][...]
        crow = jax.lax.broadcasted_iota(jnp.int32, (_CH, _COLS), 0)
        ccol = jax.lax.broadcasted_iota(jnp.int32, (_CH, _COLS), 1)
        keep_oe = (((crow // _L) ^ ccol) & 1) == 0
        zero = jnp.zeros_like(x)
        mobufs[c % 2][...] = jnp.where(keep_oe, x, zero)
        mebufs[c % 2][...] = jnp.where(keep_oe, zero, x)
        moe_copy(c).start()
        meo_copy(c).start()

    moe_copy(_NCH - 2).wait()
    meo_copy(_NCH - 2).wait()
    moe_copy(_NCH - 1).wait()
    meo_copy(_NCH - 1).wait()
    for k in range(_NPAT):
        pltpu.make_async_copy(
            oe_pat, oe_hbm.at[pl.ds(k * _PAT, _PAT), :], sem_mask).wait()
        pltpu.make_async_copy(
            eo_pat, eo_hbm.at[pl.ds(k * _PAT, _PAT), :], sem_mask).wait()


def kernel(inputs):
    x2d = jnp.transpose(inputs, (0, 1, 3, 2)).reshape(_ROWS, _COLS)
    any_spec = pl.BlockSpec(memory_space=pl.ANY)
    out = pl.pallas_call(
        _mask_kernel,
        in_specs=[any_spec],
        out_specs=[any_spec, any_spec, any_spec, any_spec],
        out_shape=[
            jax.ShapeDtypeStruct((_ROWS, _COLS), jnp.float32),
            jax.ShapeDtypeStruct((_ROWS, _COLS), jnp.float32),
            jax.ShapeDtypeStruct((_ROWS, _COLS), jnp.int32),
            jax.ShapeDtypeStruct((_ROWS, _COLS), jnp.int32),
        ],
        scratch_shapes=[
            pltpu.VMEM((_CH, _COLS), jnp.float32),
            pltpu.VMEM((_CH, _COLS), jnp.float32),
            pltpu.VMEM((_CH, _COLS), jnp.float32),
            pltpu.VMEM((_CH, _COLS), jnp.float32),
            pltpu.VMEM((_CH, _COLS), jnp.float32),
            pltpu.VMEM((_CH, _COLS), jnp.float32),
            pltpu.VMEM((_CH, _COLS), jnp.float32),
            pltpu.VMEM((_CH, _COLS), jnp.float32),
            pltpu.VMEM((_PAT, _COLS), jnp.int32),
            pltpu.VMEM((_PAT, _COLS), jnp.int32),
            pltpu.SemaphoreType.DMA,
            pltpu.SemaphoreType.DMA,
            pltpu.SemaphoreType.DMA,
            pltpu.SemaphoreType.DMA,
        ],
    )(x2d)

    def _back(a):
        return jnp.transpose(a.reshape(_B, _C, _L, _P), (0, 1, 3, 2))

    return tuple(_back(a) for a in out)


# speedup vs baseline: 1.0295x; 1.0008x over previous
import jax
import jax.numpy as jnp
from jax.experimental import pallas as pl
from jax.experimental.pallas import tpu as pltpu

_B = 32
_C = 64
_P = 128
_L = 16
_COLS = _P
_ROWS = _B * _C * _L
_CH = 8192
_NCH = _ROWS // _CH
_PAT = 8192
_NPAT = _ROWS // _PAT


def _mask_kernel(x_hbm, moe_hbm, meo_hbm, oe_hbm, eo_hbm,
                 xb0, xb1, xb2, xb3, mo0, mo1, me0, me1, oe_pat, eo_pat,
                 sem_in, sem_moe, sem_meo, sem_mask, sem_mask2):
    xbufs = (xb0, xb1, xb2, xb3)
    mobufs = (mo0, mo1)
    mebufs = (me0, me1)

    def in_copy(c):
        return pltpu.make_async_copy(
            x_hbm.at[pl.ds(c * _CH, _CH), :], xbufs[c], sem_in)

    def moe_copy(c):
        return pltpu.make_async_copy(
            mobufs[c % 2], moe_hbm.at[pl.ds(c * _CH, _CH), :], sem_moe)

    def meo_copy(c):
        return pltpu.make_async_copy(
            mebufs[c % 2], meo_hbm.at[pl.ds(c * _CH, _CH), :], sem_meo)

    for c in range(_NCH):
        in_copy(c).start()

    prow = jax.lax.broadcasted_iota(jnp.int32, (_PAT, _COLS), 0)
    pcol = jax.lax.broadcasted_iota(jnp.int32, (_PAT, _COLS), 1)
    poe = ((prow // _L) ^ pcol) & 1
    oe_pat[...] = poe
    eo_pat[...] = poe ^ 1
    for k in range(_NPAT):
        pltpu.make_async_copy(
            oe_pat, oe_hbm.at[pl.ds(k * _PAT, _PAT), :], sem_mask).start()
        pltpu.make_async_copy(
            eo_pat, eo_hbm.at[pl.ds(k * _PAT, _PAT), :], sem_mask2).start()

    for c in range(_NCH):
        in_copy(c).wait()
        if c >= 2:
            moe_copy(c - 2).wait()
            meo_copy(c - 2).wait()
        x = xbufs[c][...]
        crow = jax.lax.broadcasted_iota(jnp.int32, (_CH, _COLS), 0)
        ccol = jax.lax.broadcasted_iota(jnp.int32, (_CH, _COLS), 1)
        keep_oe = (((crow // _L) ^ ccol) & 1) == 0
        zero = jnp.zeros_like(x)
        mobufs[c % 2][...] = jnp.where(keep_oe, x, zero)
        mebufs[c % 2][...] = jnp.where(keep_oe, zero, x)
        moe_copy(c).start()
        meo_copy(c).start()

    moe_copy(_NCH - 2).wait()
    meo_copy(_NCH - 2).wait()
    moe_copy(_NCH - 1).wait()
    meo_copy(_NCH - 1).wait()
    for k in range(_NPAT):
        pltpu.make_async_copy(
            oe_pat, oe_hbm.at[pl.ds(k * _PAT, _PAT), :], sem_mask).wait()
        pltpu.make_async_copy(
            eo_pat, eo_hbm.at[pl.ds(k * _PAT, _PAT), :], sem_mask2).wait()


def kernel(inputs):
    x2d = jnp.transpose(inputs, (0, 1, 3, 2)).reshape(_ROWS, _COLS)
    any_spec = pl.BlockSpec(memory_space=pl.ANY)
    out = pl.pallas_call(
        _mask_kernel,
        in_specs=[any_spec],
        out_specs=[any_spec, any_spec, any_spec, any_spec],
        out_shape=[
            jax.ShapeDtypeStruct((_ROWS, _COLS), jnp.float32),
            jax.ShapeDtypeStruct((_ROWS, _COLS), jnp.float32),
            jax.ShapeDtypeStruct((_ROWS, _COLS), jnp.int32),
            jax.ShapeDtypeStruct((_ROWS, _COLS), jnp.int32),
        ],
        scratch_shapes=[
            pltpu.VMEM((_CH, _COLS), jnp.float32),
            pltpu.VMEM((_CH, _COLS), jnp.float32),
            pltpu.VMEM((_CH, _COLS), jnp.float32),
            pltpu.VMEM((_CH, _COLS), jnp.float32),
            pltpu.VMEM((_CH, _COLS), jnp.float32),
            pltpu.VMEM((_CH, _COLS), jnp.float32),
            pltpu.VMEM((_CH, _COLS), jnp.float32),
            pltpu.VMEM((_CH, _COLS), jnp.float32),
            pltpu.VMEM((_PAT, _COLS), jnp.int32),
            pltpu.VMEM((_PAT, _COLS), jnp.int32),
            pltpu.SemaphoreType.DMA,
            pltpu.SemaphoreType.DMA,
            pltpu.SemaphoreType.DMA,
            pltpu.SemaphoreType.DMA,
            pltpu.SemaphoreType.DMA,
        ],
    )(x2d)

    def _back(a):
        return jnp.transpose(a.reshape(_B, _C, _L, _P), (0, 1, 3, 2))

    return tuple(_back(a) for a in out)
